# baseline (speedup 1.0000x reference)
import functools
import math

import jax
import jax.numpy as jnp
from jax import lax
from jax.experimental import pallas as pl
from jax.experimental.pallas import tpu as pltpu
from jax.experimental.pallas import tpu_sc as plsc

_M = 4
_COS_M = math.cos(_M)
_SIN_M = math.sin(_M)

_LN = 16
_CH = 10000
_UNR = 5


def _sc_reduce(x, lbl_i32):
    b, c = x.shape
    info = plsc.get_sparse_core_info()
    nw = info.num_cores * info.num_subcores
    rpw = b // nw
    cpr = c // _CH
    nt = rpw * cpr
    nv = _CH // _LN
    ni = nv // _UNR
    assert c % _CH == 0 and nv % _UNR == 0 and nt % 2 == 0

    mesh = plsc.VectorSubcoreMesh(core_axis_name="c", subcore_axis_name="s")

    @functools.partial(
        pl.kernel,
        mesh=mesh,
        out_type=(
            jax.ShapeDtypeStruct((nw, _LN), jnp.float32),
            jax.ShapeDtypeStruct((b, _LN), jnp.float32),
        ),
        scratch_types=[
            pltpu.VMEM((_CH,), jnp.float32),
            pltpu.VMEM((_CH,), jnp.float32),
            pltpu.VMEM((rpw,), jnp.int32),
            pltpu.VMEM((rpw, _LN), jnp.float32),
            pltpu.VMEM((_LN,), jnp.float32),
            pltpu.SemaphoreType.DMA,
            pltpu.SemaphoreType.DMA,
            pltpu.SemaphoreType.DMA,
        ],
        compiler_params=pltpu.CompilerParams(use_tc_tiling_on_sc=False, needs_layout_passes=False),
    )
    def k(x_hbm, lbl_hbm, part_hbm, rows_hbm,
          buf0, buf1, lblv, gbuf, partv, sem0, sem1, semg):
        wid = lax.axis_index("s") * info.num_cores + lax.axis_index("c")
        r0 = wid * rpw

        pltpu.sync_copy(lbl_hbm.at[pl.ds(r0, rpw)], lblv)
        lane_iota = lax.broadcasted_iota(jnp.int32, (_LN,), 0)
        gds = []
        for t in range(rpw):
            vec = lblv[pl.ds((t // _LN) * _LN, _LN)]
            sel = jnp.where(lane_iota == (t % _LN), vec, 0)
            s = jnp.max(sel)
            col0 = (s // _LN) * _LN
            gds.append(pltpu.async_copy(
                x_hbm.at[r0 + t].at[pl.ds(col0, _LN)], gbuf.at[t], semg))

        def start(u, buf, sem):
            row = r0 + u // cpr
            cc = (u % cpr) * _CH
            pltpu.async_copy(x_hbm.at[row].at[pl.ds(cc, _CH)], buf, sem)

        def wait(buf, sem):
            pltpu.make_async_copy(x_hbm.at[r0].at[pl.ds(0, _CH)], buf,
                                  sem).wait()

        def compute(buf, acc):
            def inner(i, accs):
                base = i * (_LN * _UNR)
                out = []
                for u in range(_UNR):
                    v = buf[pl.ds(base + u * _LN, _LN)]
                    out.append(accs[u] + v * v)
                return tuple(out)
            accs = lax.fori_loop(
                0, ni, inner,
                tuple(jnp.zeros((_LN,), jnp.float32) for _ in range(_UNR)))
            for a in accs:
                acc = acc + a
            return acc

        start(0, buf0, sem0)
        start(1, buf1, sem1)

        def outer(i, acc):
            t0 = i * 2
            wait(buf0, sem0)
            acc = compute(buf0, acc)

            @pl.when(t0 + 2 < nt)
            def _():
                start(t0 + 2, buf0, sem0)

            wait(buf1, sem1)
            acc = compute(buf1, acc)

            @pl.when(t0 + 3 < nt)
            def _():
                start(t0 + 3, buf1, sem1)

            return acc

        acc = lax.fori_loop(0, nt // 2, outer, jnp.zeros((_LN,), jnp.float32))

        for d in gds:
            d.wait()
        partv[...] = acc
        pltpu.sync_copy(partv, part_hbm.at[wid])
        pltpu.sync_copy(gbuf, rows_hbm.at[pl.ds(r0, rpw)])

    return k(x, lbl_i32)


def _tc_fin(part_ref, rows_ref, lbl_ref, out_ref, *, n):
    total = jnp.sum(part_ref[...])
    rows = rows_ref[...]
    lane = jax.lax.broadcasted_iota(jnp.int32, rows.shape, 1)
    off = lbl_ref[...] % _LN
    v = jnp.sum(jnp.where(lane == off, rows, 0.0), axis=1, keepdims=True)
    phi = v * _COS_M - jnp.sqrt(jnp.maximum(1.0 - v * v, 0.0)) * _SIN_M
    corr = jnp.sum(phi * phi - v * v)
    out_ref[0, 0, 0] = (total + corr) * (_M * _M / n)


def kernel(input, label):
    b, c = input.shape
    lbl = label.astype(jnp.int32)
    part, rows = _sc_reduce(input, lbl)

    out = pl.pallas_call(
        functools.partial(_tc_fin, n=b * c),
        out_specs=pl.BlockSpec((1, 1, 1), lambda: (0, 0, 0),
                               memory_space=pltpu.SMEM),
        out_shape=jax.ShapeDtypeStruct((1, 1, 1), jnp.float32),
    )(part, rows, lbl.reshape(b, 1))
    return out.reshape(())
